# baseline (speedup 1.0000x reference)
import functools

import jax
import jax.numpy as jnp
from jax import lax
from jax.experimental import pallas as pl
from jax.experimental.pallas import tpu as pltpu
from jax.experimental.pallas import tpu_sc as plsc

B = 1024
I = 100
D = 128
T = 3
E = 320000
NU = 10000
NI = 10000
L2N = 1e-4
EPS = 1e-8

NC = 2
NS = 16
NW = NC * NS
EPT = E // NW
AGR = 1152
LUTN = NU + 16
SLN = EPT + 288
NCH = 81
NB = 2048
NIP = 10240
BPT = B // NW

_mesh = plsc.VectorSubcoreMesh(
    core_axis_name="c", subcore_axis_name="s", num_cores=NC, num_subcores=NS)


@functools.partial(
    pl.kernel,
    mesh=_mesh,
    out_type=(
        jax.ShapeDtypeStruct((NC, T, B, D), jnp.float32),
        jax.ShapeDtypeStruct((B, D), jnp.float32),
        jax.ShapeDtypeStruct((B, 128), jnp.float32),
    ),
    scratch_types=[
        pltpu.VMEM((LUTN,), jnp.int32),
        pltpu.VMEM((B,), jnp.int32),
        pltpu.VMEM((EPT,), jnp.int32),
        pltpu.VMEM((EPT,), jnp.int32),
        pltpu.VMEM((SLN,), jnp.int32),
        pltpu.VMEM((NCH, 128), jnp.int32),
        pltpu.VMEM((SLN,), jnp.int32),
        pltpu.VMEM((256, D), jnp.float32),
        pltpu.VMEM((BPT, D), jnp.float32),
        pltpu.VMEM_SHARED((T * AGR, D), jnp.float32),
        pltpu.SemaphoreType.DMA,
        pltpu.SemaphoreType.DMA,
        pltpu.SemaphoreType.DMA,
        pltpu.SemaphoreType.DMA,
    ],
    compiler_params=pltpu.CompilerParams(needs_layout_passes=False),
)
def _sc_aggregate(users_h, relu_h, reli_h, se_h, ue_h, degp_h, zeros_h,
                  vraw_h, urows_h, degrows_h,
                  lut_v, users_v, ru_v, ri_v, slf_v, sl2_v, itf_v, rows_v,
                  ub_v, agg_sh, sem, sem2, ssem, ssem2):
    c = lax.axis_index("c")
    s = lax.axis_index("s")
    w = s * NC + c

    zrows = (T * AGR) // NS
    pltpu.sync_copy(zeros_h.at[pl.ds(s * zrows, zrows)],
                    agg_sh.at[pl.ds(s * zrows, zrows)])
    pltpu.sync_copy(users_h, users_v)

    def _init(i, carry):
        lut_v[pl.ds(i * 16, 16)] = jnp.full((16,), -1, jnp.int32)
        return carry
    lax.fori_loop(0, LUTN // 16, _init, 0)

    lanes = lax.iota(jnp.int32, 16)

    def _scat(i, carry):
        idx = users_v[pl.ds(i * 16, 16)]
        plsc.store_scatter(lut_v, [idx], i * 16 + lanes)
        return carry
    lax.fori_loop(0, B // 16, _scat, 0)

    plsc.subcore_barrier()

    for t in range(T):
        pltpu.sync_copy(relu_h.at[pl.ds(t * E + w * EPT, EPT)], ru_v)
        pltpu.sync_copy(reli_h.at[pl.ds(t * E + w * EPT, EPT)], ri_v)

        def _filt(i, cnt):
            ru = ru_v[pl.ds(i * 16, 16)]
            ri = ri_v[pl.ds(i * 16, 16)]
            sl = plsc.load_gather(lut_v, [ru])
            m = sl >= 0
            plsc.store_compressed(slf_v.at[pl.ds(cnt, 16)], sl + t * AGR,
                                  mask=m)
            plsc.store_compressed(itf_v.at[pl.ds(cnt, 16)], ri + t * NI,
                                  mask=m)
            return cnt + jnp.sum(m.astype(jnp.int32))
        cnt = lax.fori_loop(0, EPT // 16, _filt, jnp.int32(0))

        dummy_sl = jnp.full((16,), t * AGR + B, jnp.int32)
        zero_it = jnp.zeros((16,), jnp.int32)
        for k in range(16):
            slf_v[pl.ds(cnt + k * 16, 16)] = dummy_sl
            itf_v[pl.ds(cnt + k * 16, 16)] = zero_it

        nch2 = (cnt + 255) // 256
        nch = nch2 * 2

        def _c2d(j, carry):
            for k in range(8):
                sl2_v[j, pl.ds(k * 16, 16)] = slf_v[pl.ds(j * 128 + k * 16, 16)]
            return carry
        lax.fori_loop(0, nch, _c2d, 0)

        def _chunk(j, carry):
            pltpu.async_copy(se_h.at[itf_v.at[pl.ds(j * 256, 256)]],
                             rows_v, sem).wait()
            for k in range(2):
                pltpu.make_async_copy(
                    rows_v.at[pl.ds(k * 128, 128)],
                    agg_sh.at[sl2_v.at[2 * j + k]], ssem).start(add=True)
            for k in range(2):
                pltpu.make_async_copy(
                    rows_v.at[pl.ds(k * 128, 128)],
                    agg_sh.at[sl2_v.at[2 * j + k]], ssem).wait()
            return carry
        lax.fori_loop(0, nch2, _chunk, 0)

    base = w * BPT
    pltpu.async_copy(ue_h.at[users_v.at[pl.ds(base, BPT)]], ub_v, sem).wait()
    pltpu.sync_copy(ub_v, urows_h.at[pl.ds(base, BPT)])
    pltpu.async_copy(degp_h.at[users_v.at[pl.ds(base, BPT)]], ub_v, sem).wait()
    pltpu.sync_copy(ub_v, degrows_h.at[pl.ds(base, BPT)])

    plsc.subcore_barrier()

    rpt = B // NS
    for t in range(T):
        pltpu.sync_copy(agg_sh.at[pl.ds(t * AGR + s * rpt, rpt)],
                        vraw_h.at[c, t, pl.ds(s * rpt, rpt)])


def _tc_body(users_c_ref, users_r_ref, vraw_ref, deg_ref, w_ref, u_ref,
             ie_ref, s0_ref, s1_ref, s2_ref,
             out_ref, nrm_ref, l2u_ref, us_s):
    i = pl.program_id(0)

    @pl.when(i == 0)
    def _prologue():
        vr = vraw_ref[0] + vraw_ref[1]
        g = (users_c_ref[...] == users_r_ref[...]).astype(jnp.float32)
        for t in range(T):
            vt = lax.dot_general(g, vr[t], (((1,), (0,)), ((), ())),
                                 preferred_element_type=jnp.float32)
            vt = vt / (deg_ref[:, t:t + 1] + EPS)
            mt = lax.dot_general(w_ref[t], w_ref[t], (((1,), (1,)), ((), ())),
                                 preferred_element_type=jnp.float32)
            us_s[t] = lax.dot_general(vt, mt, (((1,), (0,)), ((), ())),
                                      preferred_element_type=jnp.float32) * (1.0 / 6.0)
        uu = u_ref[...]
        l2u_ref[...] = ((L2N * I) * jnp.sum(uu * uu)).reshape(1, 1)

    ie = ie_ref[...]
    blk = 0.5 * lax.dot_general(u_ref[...], ie, (((1,), (1,)), ((), ())),
                                preferred_element_type=jnp.float32)
    for t, sref in enumerate((s0_ref, s1_ref, s2_ref)):
        blk = blk + lax.dot_general(us_s[t], sref[...],
                                    (((1,), (1,)), ((), ())),
                                    preferred_element_type=jnp.float32)
    out_ref[...] = blk
    nrm_ref[...] = jnp.broadcast_to(
        jnp.sum(ie * ie, axis=1).reshape(1, NB), (8, NB))


def _tc_dense(users_c, users_r, vraw, deg_rows, w_scores, u_rows,
              item_embedding, s0, s1, s2):
    grid = (NIP // NB,)
    cst = lambda i: (0, 0)
    cst3 = lambda i: (0, 0, 0)
    blk = lambda i: (i, 0)
    return pl.pallas_call(
        _tc_body,
        grid=grid,
        in_specs=[
            pl.BlockSpec((B, 1), cst),
            pl.BlockSpec((1, B), cst),
            pl.BlockSpec((NC, T, B, D), lambda i: (0, 0, 0, 0)),
            pl.BlockSpec((B, 128), cst),
            pl.BlockSpec((T, D, D), cst3),
            pl.BlockSpec((B, D), cst),
            pl.BlockSpec((NB, D), blk),
            pl.BlockSpec((NB, D), blk),
            pl.BlockSpec((NB, D), blk),
            pl.BlockSpec((NB, D), blk),
        ],
        out_specs=[
            pl.BlockSpec((B, NB), lambda i: (0, i)),
            pl.BlockSpec((8, NB), lambda i: (0, i)),
            pl.BlockSpec((1, 1), cst),
        ],
        out_shape=[
            jax.ShapeDtypeStruct((B, NIP), jnp.float32),
            jax.ShapeDtypeStruct((8, NIP), jnp.float32),
            jax.ShapeDtypeStruct((1, 1), jnp.float32),
        ],
        scratch_shapes=[pltpu.VMEM((T, B, D), jnp.float32)],
        compiler_params=pltpu.CompilerParams(
            dimension_semantics=("arbitrary",)),
    )(users_c, users_r, vraw, deg_rows, w_scores, u_rows,
      item_embedding, s0, s1, s2)


@functools.partial(
    pl.kernel,
    mesh=_mesh,
    out_type=(
        jax.ShapeDtypeStruct((B * 104,), jnp.float32),
        jax.ShapeDtypeStruct((NC * 16,), jnp.float32),
    ),
    scratch_types=[
        pltpu.VMEM((NIP,), jnp.float32),
        pltpu.VMEM((NIP,), jnp.float32),
        pltpu.VMEM((BPT * I + 16,), jnp.int32),
        pltpu.VMEM((112,), jnp.float32),
        pltpu.VMEM((16,), jnp.float32),
        pltpu.VMEM((NS * 16,), jnp.float32),
        pltpu.VMEM_SHARED((NS * 16,), jnp.float32),
    ],
    compiler_params=pltpu.CompilerParams(needs_layout_passes=False),
)
def _sc_takealong(sf_h, nrm_h, items_h, out_h, l2i_h,
                  srow_v, nrm_v, it_v, orow_v, acc_v, red_v, l2_sh):
    c = lax.axis_index("c")
    s = lax.axis_index("s")
    w = s * NC + c

    pltpu.sync_copy(nrm_h.at[0], nrm_v)
    it_v[pl.ds(BPT * I, 16)] = jnp.zeros((16,), jnp.int32)
    pltpu.sync_copy(items_h.at[pl.ds(w * BPT * I, BPT * I)],
                    it_v.at[pl.ds(0, BPT * I)])

    lanes = lax.iota(jnp.int32, 16)
    acc = jnp.zeros((16,), jnp.float32)
    for r in range(BPT):
        b = w * BPT + r
        pltpu.sync_copy(sf_h.at[pl.ds(b * NIP, NIP)], srow_v)
        for k in range(7):
            idx = it_v[pl.ds(r * I + k * 16, 16)]
            sv = plsc.load_gather(srow_v, [idx])
            nv = plsc.load_gather(nrm_v, [idx])
            orow_v[pl.ds(k * 16, 16)] = sv
            if k == 6:
                nv = jnp.where(lanes < 4, nv, 0.0)
            acc = acc + nv
        pltpu.sync_copy(orow_v.at[pl.ds(0, 104)], out_h.at[pl.ds(b * 104, 104)])

    acc_v[...] = acc
    pltpu.sync_copy(acc_v, l2_sh.at[pl.ds(s * 16, 16)])
    plsc.subcore_barrier()

    @pl.when(s == 0)
    def _reduce():
        pltpu.sync_copy(l2_sh, red_v)
        tot = jnp.zeros((16,), jnp.float32)
        for r in range(NS):
            tot = tot + red_v[pl.ds(r * 16, 16)]
        acc_v[...] = jnp.broadcast_to(jnp.sum(tot), (16,))
        pltpu.sync_copy(acc_v, l2i_h.at[pl.ds(c * 16, 16)])


def kernel(users, items, user_embedding, item_embedding, s_embs, W_scores,
           rel_user, rel_item, user_behavior_degree):
    users = users.astype(jnp.int32)
    items = items.astype(jnp.int32)
    rel_user = rel_user.astype(jnp.int32)
    rel_item = rel_item.astype(jnp.int32)

    se_flat = s_embs.reshape(T * NI, D)
    degp = jnp.pad(user_behavior_degree, ((0, 0), (0, 125)))
    zeros = jnp.zeros((T * AGR, D), jnp.float32)

    vraw, u_rows, deg_rows = _sc_aggregate(
        users, rel_user.reshape(T * E), rel_item.reshape(T * E),
        se_flat, user_embedding, degp, zeros)

    scores_full, normie, l2u = _tc_dense(
        users.reshape(B, 1), users.reshape(1, B), vraw, deg_rows, W_scores,
        u_rows, item_embedding, s_embs[0], s_embs[1], s_embs[2])

    out_flat, l2i = _sc_takealong(scores_full.reshape(B * NIP), normie,
                                  items.reshape(B * I))

    scores = out_flat.reshape(B, 104)[:, :I]
    l2 = l2u[0, 0] + L2N * (l2i[0] + l2i[16])
    return scores, l2


# speedup vs baseline: 1.2882x; 1.2882x over previous
import functools

import jax
import jax.numpy as jnp
from jax import lax
from jax.experimental import pallas as pl
from jax.experimental.pallas import tpu as pltpu
from jax.experimental.pallas import tpu_sc as plsc

B = 1024
I = 100
D = 128
T = 3
E = 320000
NU = 10000
NI = 10000
L2N = 1e-4
EPS = 1e-8

NC = 2
NS = 16
NW = NC * NS
EPT = E // NW
AGR = 1152
LUTN = NU + 16
SLN = EPT + 144
NCH = 80
NB = 2048
NIP = 10240
BPT = B // NW

_mesh = plsc.VectorSubcoreMesh(
    core_axis_name="c", subcore_axis_name="s", num_cores=NC, num_subcores=NS)


@functools.partial(
    pl.kernel,
    mesh=_mesh,
    out_type=(
        jax.ShapeDtypeStruct((NC, T, B, D), jnp.float32),
        jax.ShapeDtypeStruct((B, D), jnp.float32),
        jax.ShapeDtypeStruct((B, 128), jnp.float32),
    ),
    scratch_types=[
        pltpu.VMEM((LUTN,), jnp.int32),
        pltpu.VMEM((B,), jnp.int32),
        pltpu.VMEM((EPT,), jnp.int32),
        pltpu.VMEM((EPT,), jnp.int32),
        pltpu.VMEM((SLN,), jnp.int32),
        pltpu.VMEM((NCH, 128), jnp.int32),
        pltpu.VMEM((SLN,), jnp.int32),
        pltpu.VMEM((128, D), jnp.float32),
        pltpu.VMEM((128, D), jnp.float32),
        pltpu.VMEM((BPT, D), jnp.float32),
        pltpu.VMEM((BPT, 128), jnp.float32),
        pltpu.VMEM_SHARED((T * AGR, D), jnp.float32),
        pltpu.SemaphoreType.DMA,
        pltpu.SemaphoreType.DMA,
        pltpu.SemaphoreType.DMA,
        pltpu.SemaphoreType.DMA,
    ],
    compiler_params=pltpu.CompilerParams(needs_layout_passes=False),
)
def _sc_aggregate(users_h, relu_h, reli_h, se_h, ue_h, degp_h, zeros_h,
                  vraw_h, urows_h, degrows_h,
                  lut_v, users_v, ru_v, ri_v, slf_v, sl2_v, itf_v, rows_v,
                  rows2_v, ub_v, db_v, agg_sh, sem, sem2, ssem, ssem2):
    c = lax.axis_index("c")
    s = lax.axis_index("s")
    w = s * NC + c

    zrows = (T * AGR) // NS
    pltpu.sync_copy(zeros_h.at[pl.ds(s * zrows, zrows)],
                    agg_sh.at[pl.ds(s * zrows, zrows)])
    pltpu.sync_copy(users_h, users_v)

    def _init(i, carry):
        lut_v[pl.ds(i * 16, 16)] = jnp.full((16,), -1, jnp.int32)
        return carry
    lax.fori_loop(0, LUTN // 16, _init, 0)

    lanes = lax.iota(jnp.int32, 16)

    def _scat(i, carry):
        idx = users_v[pl.ds(i * 16, 16)]
        plsc.store_scatter(lut_v, [idx], i * 16 + lanes)
        return carry
    lax.fori_loop(0, B // 16, _scat, 0)

    plsc.subcore_barrier()

    for t in range(T):
        pltpu.sync_copy(relu_h.at[pl.ds(t * E + w * EPT, EPT)], ru_v)
        pltpu.sync_copy(reli_h.at[pl.ds(t * E + w * EPT, EPT)], ri_v)

        def _filt(i, cnt):
            ru = ru_v[pl.ds(i * 16, 16)]
            ri = ri_v[pl.ds(i * 16, 16)]
            sl = plsc.load_gather(lut_v, [ru])
            m = sl >= 0
            plsc.store_compressed(slf_v.at[pl.ds(cnt, 16)], sl + t * AGR,
                                  mask=m)
            plsc.store_compressed(itf_v.at[pl.ds(cnt, 16)], ri, mask=m)
            return cnt + jnp.sum(m.astype(jnp.int32))
        cnt = lax.fori_loop(0, EPT // 16, _filt, jnp.int32(0))

        dummy_sl = jnp.full((16,), t * AGR + B, jnp.int32)
        zero_it = jnp.zeros((16,), jnp.int32)
        for k in range(8):
            slf_v[pl.ds(cnt + k * 16, 16)] = dummy_sl
            itf_v[pl.ds(cnt + k * 16, 16)] = zero_it

        nch = (cnt + 127) // 128

        def _c2d(j, carry):
            for k in range(8):
                sl2_v[j, pl.ds(k * 16, 16)] = slf_v[pl.ds(j * 128 + k * 16, 16)]
            return carry
        lax.fori_loop(0, nch, _c2d, 0)

        def _pair(p, carry):
            j0 = 2 * p
            j1 = j0 + 1
            g0 = pltpu.make_async_copy(
                se_h.at[t].at[itf_v.at[pl.ds(j0 * 128, 128)]], rows_v, sem)
            g0.start()

            @pl.when(j1 < nch)
            def _g1():
                pltpu.make_async_copy(
                    se_h.at[t].at[itf_v.at[pl.ds(j1 * 128, 128)]], rows2_v,
                    sem2).start()

            g0.wait()
            s0 = pltpu.make_async_copy(rows_v, agg_sh.at[sl2_v.at[j0]], ssem)
            s0.start(add=True)

            @pl.when(j1 < nch)
            def _s1():
                pltpu.make_async_copy(
                    se_h.at[t].at[itf_v.at[pl.ds(j1 * 128, 128)]], rows2_v,
                    sem2).wait()
                pltpu.make_async_copy(rows2_v, agg_sh.at[sl2_v.at[j1]],
                                      ssem2).start(add=True)

            s0.wait()

            @pl.when(j1 < nch)
            def _s1w():
                pltpu.make_async_copy(rows2_v, agg_sh.at[sl2_v.at[j1]],
                                      ssem2).wait()

            return carry
        lax.fori_loop(0, (nch + 1) // 2, _pair, 0)

    base = w * BPT
    pltpu.async_copy(ue_h.at[users_v.at[pl.ds(base, BPT)]], ub_v, sem).wait()
    pltpu.sync_copy(ub_v, urows_h.at[pl.ds(base, BPT)])
    pltpu.async_copy(degp_h.at[users_v.at[pl.ds(base, BPT)]], db_v, sem).wait()
    pltpu.sync_copy(db_v, degrows_h.at[pl.ds(base, BPT)])

    plsc.subcore_barrier()

    rpt = B // NS
    for t in range(T):
        pltpu.sync_copy(agg_sh.at[pl.ds(t * AGR + s * rpt, rpt)],
                        vraw_h.at[c, t, pl.ds(s * rpt, rpt)])


def _tc_body(users_c_ref, users_r_ref, vraw_ref, deg_ref, w_ref, u_ref,
             ie_ref, s0_ref, s1_ref, s2_ref,
             out_ref, nrm_ref, l2u_ref, us_s):
    i = pl.program_id(0)

    @pl.when(i == 0)
    def _prologue():
        vr = vraw_ref[0] + vraw_ref[1]
        g = (users_c_ref[...] == users_r_ref[...]).astype(jnp.float32)
        for t in range(T):
            vt = lax.dot_general(g, vr[t], (((1,), (0,)), ((), ())),
                                 preferred_element_type=jnp.float32)
            vt = vt / (deg_ref[:, t:t + 1] + EPS)
            mt = lax.dot_general(w_ref[t], w_ref[t], (((1,), (1,)), ((), ())),
                                 preferred_element_type=jnp.float32)
            us_s[t] = lax.dot_general(vt, mt, (((1,), (0,)), ((), ())),
                                      preferred_element_type=jnp.float32) * (1.0 / 6.0)
        uu = u_ref[...]
        l2u_ref[...] = ((L2N * I) * jnp.sum(uu * uu)).reshape(1, 1)

    ie = ie_ref[...]
    blk = 0.5 * lax.dot_general(u_ref[...], ie, (((1,), (1,)), ((), ())),
                                preferred_element_type=jnp.float32)
    for t, sref in enumerate((s0_ref, s1_ref, s2_ref)):
        blk = blk + lax.dot_general(us_s[t], sref[...],
                                    (((1,), (1,)), ((), ())),
                                    preferred_element_type=jnp.float32)
    out_ref[...] = blk
    nrm_ref[...] = jnp.broadcast_to(
        jnp.sum(ie * ie, axis=1).reshape(1, NB), (8, NB))


def _tc_dense(users_c, users_r, vraw, deg_rows, w_scores, u_rows,
              item_embedding, s0, s1, s2):
    grid = (NIP // NB,)
    cst = lambda i: (0, 0)
    cst3 = lambda i: (0, 0, 0)
    blk = lambda i: (i, 0)
    return pl.pallas_call(
        _tc_body,
        grid=grid,
        in_specs=[
            pl.BlockSpec((B, 1), cst),
            pl.BlockSpec((1, B), cst),
            pl.BlockSpec((NC, T, B, D), lambda i: (0, 0, 0, 0)),
            pl.BlockSpec((B, 128), cst),
            pl.BlockSpec((T, D, D), cst3),
            pl.BlockSpec((B, D), cst),
            pl.BlockSpec((NB, D), blk),
            pl.BlockSpec((NB, D), blk),
            pl.BlockSpec((NB, D), blk),
            pl.BlockSpec((NB, D), blk),
        ],
        out_specs=[
            pl.BlockSpec((B, NB), lambda i: (0, i)),
            pl.BlockSpec((8, NB), lambda i: (0, i)),
            pl.BlockSpec((1, 1), cst),
        ],
        out_shape=[
            jax.ShapeDtypeStruct((B, NIP), jnp.float32),
            jax.ShapeDtypeStruct((8, NIP), jnp.float32),
            jax.ShapeDtypeStruct((1, 1), jnp.float32),
        ],
        scratch_shapes=[pltpu.VMEM((T, B, D), jnp.float32)],
        compiler_params=pltpu.CompilerParams(
            dimension_semantics=("arbitrary",)),
    )(users_c, users_r, vraw, deg_rows, w_scores, u_rows,
      item_embedding, s0, s1, s2)


@functools.partial(
    pl.kernel,
    mesh=_mesh,
    out_type=(
        jax.ShapeDtypeStruct((B * 104,), jnp.float32),
        jax.ShapeDtypeStruct((NC * 16,), jnp.float32),
    ),
    scratch_types=[
        pltpu.VMEM((NIP,), jnp.float32),
        pltpu.VMEM((NIP,), jnp.float32),
        pltpu.VMEM((BPT * I + 16,), jnp.int32),
        pltpu.VMEM((112,), jnp.float32),
        pltpu.VMEM((16,), jnp.float32),
        pltpu.VMEM((NS * 16,), jnp.float32),
        pltpu.VMEM_SHARED((NS * 16,), jnp.float32),
    ],
    compiler_params=pltpu.CompilerParams(needs_layout_passes=False),
)
def _sc_takealong(sf_h, nrm_h, items_h, out_h, l2i_h,
                  srow_v, nrm_v, it_v, orow_v, acc_v, red_v, l2_sh):
    c = lax.axis_index("c")
    s = lax.axis_index("s")
    w = s * NC + c

    pltpu.sync_copy(nrm_h.at[0], nrm_v)
    it_v[pl.ds(BPT * I, 16)] = jnp.zeros((16,), jnp.int32)
    pltpu.sync_copy(items_h.at[pl.ds(w * BPT * I, BPT * I)],
                    it_v.at[pl.ds(0, BPT * I)])

    lanes = lax.iota(jnp.int32, 16)
    acc = jnp.zeros((16,), jnp.float32)
    for r in range(BPT):
        b = w * BPT + r
        pltpu.sync_copy(sf_h.at[pl.ds(b * NIP, NIP)], srow_v)
        for k in range(7):
            idx = it_v[pl.ds(r * I + k * 16, 16)]
            sv = plsc.load_gather(srow_v, [idx])
            nv = plsc.load_gather(nrm_v, [idx])
            orow_v[pl.ds(k * 16, 16)] = sv
            if k == 6:
                nv = jnp.where(lanes < 4, nv, 0.0)
            acc = acc + nv
        pltpu.sync_copy(orow_v.at[pl.ds(0, 104)], out_h.at[pl.ds(b * 104, 104)])

    acc_v[...] = acc
    pltpu.sync_copy(acc_v, l2_sh.at[pl.ds(s * 16, 16)])
    plsc.subcore_barrier()

    @pl.when(s == 0)
    def _reduce():
        pltpu.sync_copy(l2_sh, red_v)
        tot = jnp.zeros((16,), jnp.float32)
        for r in range(NS):
            tot = tot + red_v[pl.ds(r * 16, 16)]
        acc_v[...] = jnp.broadcast_to(jnp.sum(tot), (16,))
        pltpu.sync_copy(acc_v, l2i_h.at[pl.ds(c * 16, 16)])


def kernel(users, items, user_embedding, item_embedding, s_embs, W_scores,
           rel_user, rel_item, user_behavior_degree):
    users = users.astype(jnp.int32)
    items = items.astype(jnp.int32)
    rel_user = rel_user.astype(jnp.int32)
    rel_item = rel_item.astype(jnp.int32)

    degp = jnp.pad(user_behavior_degree, ((0, 0), (0, 125)))
    zeros = jnp.zeros((T * AGR, D), jnp.float32)

    vraw, u_rows, deg_rows = _sc_aggregate(
        users, rel_user.reshape(T * E), rel_item.reshape(T * E),
        s_embs, user_embedding, degp, zeros)

    scores_full, normie, l2u = _tc_dense(
        users.reshape(B, 1), users.reshape(1, B), vraw, deg_rows, W_scores,
        u_rows, item_embedding, s_embs[0], s_embs[1], s_embs[2])

    out_flat, l2i = _sc_takealong(scores_full.reshape(B * NIP), normie,
                                  items.reshape(B * I))

    scores = out_flat.reshape(B, 104)[:, :I]
    l2 = l2u[0, 0] + L2N * (l2i[0] + l2i[16])
    return scores, l2
